# Initial kernel scaffold; baseline (speedup 1.0000x reference)
import jax
import jax.numpy as jnp
from jax.experimental import pallas as pl


def kernel(x, W_E):
    raise NotImplementedError("write your pallas kernel here")


# speedup vs baseline: 3.4581x; 3.4581x over previous
import jax
import jax.numpy as jnp
from jax import lax
from jax.experimental import pallas as pl
from jax.experimental.pallas import tpu as pltpu
from jax.experimental.pallas import tpu_sc as plsc

D_MODEL = 64
D_VOCAB = 100000
BATCH = 4096
POS = 200

B_TOTAL = BATCH * POS
NW = 32
B_PER_W = B_TOTAL // NW
CHUNK = 128
N_CHUNK = B_PER_W // CHUNK

_TBLK = 4096


def _table_body(w_ref, out_ref):
    out_ref[...] = w_ref[...].T


def _build_table(W_E):
    return pl.pallas_call(
        _table_body,
        grid=(pl.cdiv(D_VOCAB, _TBLK),),
        in_specs=[pl.BlockSpec((D_MODEL, _TBLK), lambda i: (0, i))],
        out_specs=pl.BlockSpec((_TBLK, D_MODEL), lambda i: (i, 0)),
        out_shape=jax.ShapeDtypeStruct((D_VOCAB, D_MODEL), jnp.float32),
    )(W_E)


def _gather_body(idx_hbm, table_hbm, out_hbm, idx_v, rows_v, gsem):
    wid = lax.axis_index("s") * 2 + lax.axis_index("c")
    row_base = wid * N_CHUNK
    out_base = wid * B_PER_W

    pltpu.sync_copy(idx_hbm.at[pl.ds(row_base, N_CHUNK)], idx_v)

    def step(c, carry):
        pltpu.async_copy(table_hbm.at[idx_v.at[c]], rows_v, gsem).wait()
        pltpu.sync_copy(rows_v, out_hbm.at[pl.ds(out_base + c * CHUNK, CHUNK)])
        return carry

    lax.fori_loop(0, N_CHUNK, step, 0)


@jax.jit
def _embed(x, W_E):
    table = _build_table(W_E)
    idx = x.reshape(B_TOTAL // CHUNK, CHUNK).astype(jnp.int32)

    mesh = plsc.VectorSubcoreMesh(core_axis_name="c", subcore_axis_name="s")
    out = pl.kernel(
        _gather_body,
        mesh=mesh,
        out_type=jax.ShapeDtypeStruct((B_TOTAL, D_MODEL), jnp.float32),
        scratch_types=[
            pltpu.VMEM((N_CHUNK, CHUNK), jnp.int32),
            pltpu.VMEM((CHUNK, D_MODEL), jnp.float32),
            pltpu.SemaphoreType.DMA,
        ],
        compiler_params=pltpu.CompilerParams(use_tc_tiling_on_sc=False),
    )(idx, table)
    return out.reshape(BATCH, POS, D_MODEL)


def kernel(x, W_E):
    return _embed(x, W_E)


# speedup vs baseline: 4.1265x; 1.1933x over previous
import jax
import jax.numpy as jnp
from jax import lax
from jax.experimental import pallas as pl
from jax.experimental.pallas import tpu as pltpu
from jax.experimental.pallas import tpu_sc as plsc

D_MODEL = 64
D_VOCAB = 100000
BATCH = 4096
POS = 200

B_TOTAL = BATCH * POS
NW = 32
B_PER_W = B_TOTAL // NW
CHUNK = 128
N_CHUNK = B_PER_W // CHUNK

_TBLK = 4096


def _table_body(w_ref, out_ref):
    out_ref[...] = w_ref[...].T


def _build_table(W_E):
    return pl.pallas_call(
        _table_body,
        grid=(pl.cdiv(D_VOCAB, _TBLK),),
        in_specs=[pl.BlockSpec((D_MODEL, _TBLK), lambda i: (0, i))],
        out_specs=pl.BlockSpec((_TBLK, D_MODEL), lambda i: (i, 0)),
        out_shape=jax.ShapeDtypeStruct((D_VOCAB, D_MODEL), jnp.float32),
    )(W_E)


K = 4
NG = N_CHUNK // K


def _gather_body(idx_hbm, table_hbm, out_hbm, idx_v, buf_a, buf_b,
                 gsem_a, gsem_b, wsem_a, wsem_b):
    wid = lax.axis_index("s") * 2 + lax.axis_index("c")
    row_base = wid * N_CHUNK
    out_base = wid * B_PER_W

    pltpu.sync_copy(idx_hbm.at[pl.ds(row_base, N_CHUNK)], idx_v)

    bufs = (buf_a, buf_b)
    gsems = (gsem_a, gsem_b)
    wsems = (wsem_a, wsem_b)

    def fire(g, b):
        for j in range(K):
            pltpu.async_copy(
                table_hbm.at[idx_v.at[g * K + j]],
                bufs[b].at[pl.ds(j * CHUNK, CHUNK)],
                gsems[b],
            )

    def drain(g, b):
        for j in range(K):
            pltpu.make_async_copy(
                table_hbm.at[idx_v.at[g * K + j]],
                bufs[b].at[pl.ds(j * CHUNK, CHUNK)],
                gsems[b],
            ).wait()

    def write(g, b):
        pltpu.async_copy(
            bufs[b],
            out_hbm.at[pl.ds(out_base + g * K * CHUNK, K * CHUNK)],
            wsems[b],
        )

    def wait_write(g, b):
        pltpu.make_async_copy(
            bufs[b],
            out_hbm.at[pl.ds(out_base + g * K * CHUNK, K * CHUNK)],
            wsems[b],
        ).wait()

    fire(0, 0)

    def body(i, carry):
        g0 = 2 * i
        g1 = g0 + 1

        @pl.when(i > 0)
        def _():
            wait_write(g1 - 2, 1)
        fire(g1, 1)

        drain(g0, 0)
        write(g0, 0)

        @pl.when(i < NG // 2 - 1)
        def _():
            wait_write(g0, 0)
            fire(g0 + 2, 0)

        drain(g1, 1)
        write(g1, 1)
        return carry

    lax.fori_loop(0, NG // 2, body, 0)
    wait_write(NG - 2, 0)
    wait_write(NG - 1, 1)


@jax.jit
def _embed(x, W_E):
    table = _build_table(W_E)
    idx = x.reshape(B_TOTAL // CHUNK, CHUNK).astype(jnp.int32)

    mesh = plsc.VectorSubcoreMesh(core_axis_name="c", subcore_axis_name="s")
    out = pl.kernel(
        _gather_body,
        mesh=mesh,
        out_type=jax.ShapeDtypeStruct((B_TOTAL, D_MODEL), jnp.float32),
        scratch_types=[
            pltpu.VMEM((N_CHUNK, CHUNK), jnp.int32),
            pltpu.VMEM((K * CHUNK, D_MODEL), jnp.float32),
            pltpu.VMEM((K * CHUNK, D_MODEL), jnp.float32),
            pltpu.SemaphoreType.DMA,
            pltpu.SemaphoreType.DMA,
            pltpu.SemaphoreType.DMA,
            pltpu.SemaphoreType.DMA,
        ],
        compiler_params=pltpu.CompilerParams(use_tc_tiling_on_sc=False),
    )(idx, table)
    return out.reshape(BATCH, POS, D_MODEL)


def kernel(x, W_E):
    return _embed(x, W_E)
